# baseline (speedup 1.0000x reference)
import jax
import jax.numpy as jnp
from jax import lax
from jax.experimental import pallas as pl
from jax.experimental.pallas import tpu as pltpu
from jax.experimental.pallas import tpu_sc as plsc

B, S, H = 4, 2048, 768
V, T = 100000, 2
N = B * S
NC, NS, L = 2, 16, 16
NW = NC * NS
PB = 64
PH = 16
BP = 2
CH = PH * BP
NCHUNK = (PB // PH) * (B // BP)
NLG = H // L


def _emb_body(ids_hbm, tt_hbm, word_hbm, type_hbm, pos_hbm, out_hbm,
              idtmp, tttmp, idxall, ttall, wbuf0, wbuf1, wbuf2, posb,
              t0v, t1v, dvv, psem, gs0, gs1, gs2, os0, os1, os2):
    cid = lax.axis_index("c")
    sid = lax.axis_index("s")
    wid = sid * NC + cid
    pbase = wid * PB

    handles = []
    for b in range(B):
        handles.append(pltpu.async_copy(
            ids_hbm.at[pl.ds(b * S + pbase, PB)], idtmp.at[b], psem))
        handles.append(pltpu.async_copy(
            tt_hbm.at[pl.ds(b * S + pbase, PB)], tttmp.at[b], psem))
    handles.append(pltpu.async_copy(pos_hbm.at[pl.ds(pbase, PB)], posb, psem))
    handles.append(pltpu.async_copy(type_hbm.at[0], t0v, psem))
    handles.append(pltpu.async_copy(type_hbm.at[1], t1v, psem))
    for cp in handles:
        cp.wait()

    chunks = [(h, bp) for h in range(PB // PH) for bp in range(B // BP)]
    for c, (h, bp) in enumerate(chunks):
        for b2 in range(BP):
            b = bp * BP + b2
            dst = pl.ds(c * CH + b2 * PH, PH)
            idxall[dst] = idtmp[b, pl.ds(h * PH, PH)]
            ttall[dst] = tttmp[b, pl.ds(h * PH, PH)]

    wbuf = [wbuf0, wbuf1, wbuf2]
    gsem = [gs0, gs1, gs2]
    osem = [os0, os1, os2]
    NBUF = 3

    ghandles = [None] * NBUF
    out_handles = [None] * NBUF
    for k in range(NBUF - 1):
        ghandles[k] = pltpu.async_copy(
            word_hbm.at[idxall.at[pl.ds(k * CH, CH)]], wbuf[k], gsem[k])

    for l in range(NLG):
        sl = pl.ds(l * L, L)
        dvv[sl] = t1v[sl] - t0v[sl]

    for c, (h, bp) in enumerate(chunks):
        p = c % NBUF
        if c + NBUF - 1 < NCHUNK:
            pn = (c + NBUF - 1) % NBUF
            if out_handles[pn] is not None:
                for oh in out_handles[pn]:
                    oh.wait()
                out_handles[pn] = None
            ghandles[pn] = pltpu.async_copy(
                word_hbm.at[idxall.at[pl.ds((c + NBUF - 1) * CH, CH)]],
                wbuf[pn], gsem[pn])
        ghandles[p].wait()

        wb = wbuf[p]
        ttf = [ttall[pl.ds(c * CH + b2 * PH, PH)].astype(jnp.float32)
               for b2 in range(BP)]

        def l_body(l, carry, wb=wb, ttf=ttf, h=h):
            sl = pl.ds(l * L, L)
            t0 = t0v[sl]
            dv = dvv[sl]
            for r in range(PH):
                pp = posb[h * PH + r, sl] + t0
                for b2 in range(BP):
                    t = b2 * PH + r
                    wb[t, sl] = wb[t, sl] + pp + ttf[b2][r] * dv
            return carry

        lax.fori_loop(0, NLG, l_body, 0)

        out_handles[p] = []
        for b2 in range(BP):
            b = bp * BP + b2
            row0 = b * S + pbase + h * PH
            out_handles[p].append(pltpu.async_copy(
                wb.at[pl.ds(b2 * PH, PH)],
                out_hbm.at[pl.ds(row0, PH)], osem[p]))

    for hs in out_handles:
        if hs is not None:
            for oh in hs:
                oh.wait()


@jax.jit
def _emb_call(ids_flat, tt_flat, word_emb, type_emb, pos_emb):
    mesh = plsc.VectorSubcoreMesh(core_axis_name="c", subcore_axis_name="s")
    fn = pl.kernel(
        _emb_body,
        out_type=jax.ShapeDtypeStruct((N, H), jnp.float32),
        mesh=mesh,
        scratch_types=[
            pltpu.VMEM((B, PB), jnp.int32),
            pltpu.VMEM((B, PB), jnp.int32),
            pltpu.VMEM((N // NW,), jnp.int32),
            pltpu.VMEM((N // NW,), jnp.int32),
            pltpu.VMEM((CH, H), jnp.float32),
            pltpu.VMEM((CH, H), jnp.float32),
            pltpu.VMEM((CH, H), jnp.float32),
            pltpu.VMEM((PB, H), jnp.float32),
            pltpu.VMEM((H,), jnp.float32),
            pltpu.VMEM((H,), jnp.float32),
            pltpu.VMEM((H,), jnp.float32),
            pltpu.SemaphoreType.DMA,
            pltpu.SemaphoreType.DMA,
            pltpu.SemaphoreType.DMA,
            pltpu.SemaphoreType.DMA,
            pltpu.SemaphoreType.DMA,
            pltpu.SemaphoreType.DMA,
            pltpu.SemaphoreType.DMA,
        ],
    )
    return fn(ids_flat, tt_flat, word_emb, type_emb, pos_emb)


def kernel(input_ids, token_type_ids, word_emb, type_emb, pos_emb):
    ids_flat = input_ids.reshape(-1).astype(jnp.int32)
    tt_flat = token_type_ids.reshape(-1).astype(jnp.int32)
    out = _emb_call(ids_flat, tt_flat, word_emb, type_emb, pos_emb)
    return out.reshape(B, S, H)


# speedup vs baseline: 1.0356x; 1.0356x over previous
import jax
import jax.numpy as jnp
from jax import lax
from jax.experimental import pallas as pl
from jax.experimental.pallas import tpu as pltpu
from jax.experimental.pallas import tpu_sc as plsc

B, S, H = 4, 2048, 768
V, T = 100000, 2
N = B * S
NC, NS, L = 2, 16, 16
NW = NC * NS
PB = 64
PH = 16
CH = PH * B
NCHUNK = PB // PH
NLG = H // L


def _emb_body(ids_hbm, tt_hbm, word_hbm, type_hbm, pos_hbm, out_hbm,
              idtmp, tttmp, idxall, ttall, wbuf0, wbuf1, pbuf0, pbuf1,
              t0v, t1v, dvv, psem, gs0, gs1, os0, os1):
    cid = lax.axis_index("c")
    sid = lax.axis_index("s")
    wid = sid * NC + cid
    pbase = wid * PB

    handles = []
    for b in range(B):
        handles.append(pltpu.async_copy(
            ids_hbm.at[pl.ds(b * S + pbase, PB)], idtmp.at[b], psem))
        handles.append(pltpu.async_copy(
            tt_hbm.at[pl.ds(b * S + pbase, PB)], tttmp.at[b], psem))
    handles.append(pltpu.async_copy(type_hbm.at[0], t0v, psem))
    handles.append(pltpu.async_copy(type_hbm.at[1], t1v, psem))
    for cp in handles:
        cp.wait()

    for h in range(NCHUNK):
        for b in range(B):
            dst = pl.ds(h * CH + b * PH, PH)
            idxall[dst] = idtmp[b, pl.ds(h * PH, PH)]
            ttall[dst] = tttmp[b, pl.ds(h * PH, PH)]

    wbuf = [wbuf0, wbuf1]
    pbuf = [pbuf0, pbuf1]
    gsem = [gs0, gs1]
    osem = [os0, os1]

    def issue_chunk(h, p):
        g = pltpu.async_copy(
            word_hbm.at[idxall.at[pl.ds(h * CH, CH)]], wbuf[p], gsem[p])
        q = pltpu.async_copy(
            pos_hbm.at[pl.ds(pbase + h * PH, PH)], pbuf[p], gsem[p])
        return (g, q)

    ghandles = [issue_chunk(0, 0), None]
    out_handles = [None, None]

    for l in range(NLG):
        sl = pl.ds(l * L, L)
        dvv[sl] = t1v[sl] - t0v[sl]

    for h in range(NCHUNK):
        p = h & 1
        if h + 1 < NCHUNK:
            if out_handles[1 - p] is not None:
                for oh in out_handles[1 - p]:
                    oh.wait()
            ghandles[1 - p] = issue_chunk(h + 1, 1 - p)
        for gh in ghandles[p]:
            gh.wait()

        wb = wbuf[p]
        pb = pbuf[p]
        ttf = [ttall[pl.ds(h * CH + b * PH, PH)].astype(jnp.float32)
               for b in range(B)]

        def l_body(l, carry, wb=wb, pb=pb, ttf=ttf):
            sl = pl.ds(l * L, L)
            t0 = t0v[sl]
            dv = dvv[sl]
            for r in range(PH):
                pp = pb[r, sl] + t0
                for b in range(B):
                    t = b * PH + r
                    wb[t, sl] = wb[t, sl] + pp + ttf[b][r] * dv
            return carry

        lax.fori_loop(0, NLG, l_body, 0)

        out_handles[p] = []
        for b in range(B):
            row0 = b * S + pbase + h * PH
            out_handles[p].append(pltpu.async_copy(
                wb.at[pl.ds(b * PH, PH)],
                out_hbm.at[pl.ds(row0, PH)], osem[p]))

    for hs in out_handles:
        if hs is not None:
            for oh in hs:
                oh.wait()


@jax.jit
def _emb_call(ids_flat, tt_flat, word_emb, type_emb, pos_emb):
    mesh = plsc.VectorSubcoreMesh(core_axis_name="c", subcore_axis_name="s")
    fn = pl.kernel(
        _emb_body,
        out_type=jax.ShapeDtypeStruct((N, H), jnp.float32),
        mesh=mesh,
        scratch_types=[
            pltpu.VMEM((B, PB), jnp.int32),
            pltpu.VMEM((B, PB), jnp.int32),
            pltpu.VMEM((N // NW,), jnp.int32),
            pltpu.VMEM((N // NW,), jnp.int32),
            pltpu.VMEM((CH, H), jnp.float32),
            pltpu.VMEM((CH, H), jnp.float32),
            pltpu.VMEM((PH, H), jnp.float32),
            pltpu.VMEM((PH, H), jnp.float32),
            pltpu.VMEM((H,), jnp.float32),
            pltpu.VMEM((H,), jnp.float32),
            pltpu.VMEM((H,), jnp.float32),
            pltpu.SemaphoreType.DMA,
            pltpu.SemaphoreType.DMA,
            pltpu.SemaphoreType.DMA,
            pltpu.SemaphoreType.DMA,
            pltpu.SemaphoreType.DMA,
        ],
    )
    return fn(ids_flat, tt_flat, word_emb, type_emb, pos_emb)


def kernel(input_ids, token_type_ids, word_emb, type_emb, pos_emb):
    ids_flat = input_ids.reshape(-1).astype(jnp.int32)
    tt_flat = token_type_ids.reshape(-1).astype(jnp.int32)
    out = _emb_call(ids_flat, tt_flat, word_emb, type_emb, pos_emb)
    return out.reshape(B, S, H)
